# baseline (speedup 1.0000x reference)
import functools

import jax
import jax.numpy as jnp
from jax import lax
from jax.experimental import pallas as pl
from jax.experimental.pallas import tpu as pltpu
from jax.experimental.pallas import tpu_sc as plsc

NB = 4096
NT = 200
D = 64
NW = 32
BPW = NB // NW
C0 = 128
C1 = NT - C0

_mesh = plsc.VectorSubcoreMesh(core_axis_name="c", subcore_axis_name="s")


@functools.partial(
    pl.kernel,
    mesh=_mesh,
    compiler_params=pltpu.CompilerParams(use_tc_tiling_on_sc=False),
    out_type=jax.ShapeDtypeStruct((NB, NT, D), jnp.float32),
    scratch_types=[
        pltpu.VMEM((BPW, NT), jnp.int32),
        pltpu.VMEM((2, NT, D), jnp.float32),
        pltpu.SemaphoreType.DMA,
        pltpu.SemaphoreType.DMA,
    ],
)
def _emb_lookup(idx_hbm, table_hbm, out_hbm, idx_v, rows_v, in_sem, out_sem):
    wid = lax.axis_index("s") * 2 + lax.axis_index("c")
    b0 = wid * BPW
    pltpu.sync_copy(idx_hbm.at[pl.ds(b0, BPW)], idx_v)

    def row(i, db):
        @pl.when(i >= 2)
        def _():
            pltpu.make_async_copy(
                out_hbm.at[b0], rows_v.at[db], out_sem
            ).wait()

        d0 = pltpu.async_copy(
            table_hbm.at[idx_v.at[i, pl.ds(0, C0)]],
            rows_v.at[db, pl.ds(0, C0)],
            in_sem,
        )
        d1 = pltpu.async_copy(
            table_hbm.at[idx_v.at[i, pl.ds(C0, C1)]],
            rows_v.at[db, pl.ds(C0, C1)],
            in_sem,
        )
        d0.wait()
        d1.wait()
        pltpu.async_copy(rows_v.at[db], out_hbm.at[b0 + i], out_sem)

    def body(p, carry):
        row(p * 2, 0)
        row(p * 2 + 1, 1)
        return carry

    lax.fori_loop(0, BPW // 2, body, 0)
    for db in range(2):
        pltpu.make_async_copy(out_hbm.at[b0], rows_v.at[db], out_sem).wait()


def kernel(x, emb):
    return _emb_lookup(x.astype(jnp.int32), emb)
